# baseline (speedup 1.0000x reference)
import functools

import jax
import jax.numpy as jnp
from jax import lax
from jax.experimental import pallas as pl
from jax.experimental.pallas import tpu as pltpu
from jax.experimental.pallas import tpu_sc as plsc

_N = 10000
_E = 320000
_D = 128
_G = 64
_DP = 144
_NSC = 2
_NSUB = 16
_NW = _NSC * _NSUB
_EPW = _E // _NW
_CH = 40
_NCH = _EPW // _CH
_NPAD = 10240
_RPT = _NPAD // _NSUB
_RB = 1000


def _sc_agg_body(table, src, dst, zeros, out, srcb, dstb, rows, rows2, acc,
                 sem, sem2):
    c = lax.axis_index("c")
    s = lax.axis_index("s")
    wid = c * _NSUB + s

    pltpu.sync_copy(src.at[wid], srcb)
    pltpu.sync_copy(dst.at[wid], dstb)
    pltpu.sync_copy(zeros, acc.at[pl.ds(s * _RPT, _RPT)])
    plsc.subcore_barrier()

    pltpu.async_copy(table.at[srcb.at[0]], rows, sem)

    def chunkpair(i, carry):
        a = 2 * i
        pltpu.make_async_copy(table.at[srcb.at[a]], rows, sem).wait()
        pltpu.async_copy(table.at[srcb.at[a + 1]], rows2, sem2)
        pltpu.sync_copy(rows, acc.at[dstb.at[a]], add=True)
        pltpu.make_async_copy(table.at[srcb.at[a + 1]], rows2, sem2).wait()
        pltpu.async_copy(table.at[srcb.at[a + 2]], rows, sem)
        pltpu.sync_copy(rows2, acc.at[dstb.at[a + 1]], add=True)
        return carry

    lax.fori_loop(0, _NCH // 2 - 1, chunkpair, 0)
    a = _NCH - 2
    pltpu.make_async_copy(table.at[srcb.at[a]], rows, sem).wait()
    pltpu.async_copy(table.at[srcb.at[a + 1]], rows2, sem2)
    pltpu.sync_copy(rows, acc.at[dstb.at[a]], add=True)
    pltpu.make_async_copy(table.at[srcb.at[a + 1]], rows2, sem2).wait()
    pltpu.sync_copy(rows2, acc.at[dstb.at[a + 1]], add=True)
    plsc.subcore_barrier()
    pltpu.sync_copy(acc.at[pl.ds(s * _RPT, _RPT)],
                    out.at[c, pl.ds(s * _RPT, _RPT)])


@functools.cache
def _sc_agg():
    return pl.kernel(
        _sc_agg_body,
        out_type=jax.ShapeDtypeStruct((_NSC, _NPAD, _DP), jnp.float32),
        mesh=plsc.VectorSubcoreMesh(
            core_axis_name="c", subcore_axis_name="s",
            num_cores=_NSC, num_subcores=_NSUB),
        scratch_types=[
            pltpu.VMEM((_NCH, _CH), jnp.int32),
            pltpu.VMEM((_NCH, _CH), jnp.int32),
            pltpu.VMEM((_CH, _DP), jnp.float32),
            pltpu.VMEM((_CH, _DP), jnp.float32),
            pltpu.VMEM_SHARED((_NPAD, _DP), jnp.float32),
            pltpu.SemaphoreType.DMA,
            pltpu.SemaphoreType.DMA,
        ],
        compiler_params=pltpu.CompilerParams(use_tc_tiling_on_sc=False),
    )


def _ones_col(rows):
    col = lax.broadcasted_iota(jnp.int32, (rows, _DP - _D), 1)
    return jnp.where(col == 0, 1.0, 0.0).astype(jnp.float32)


def _pad_body(x_ref, o_ref):
    o_ref[...] = jnp.concatenate([x_ref[...], _ones_col(_RB)], axis=1)


_pad = pl.pallas_call(
    _pad_body,
    grid=(_N // _RB,),
    in_specs=[pl.BlockSpec((_RB, _D), lambda i: (i, 0))],
    out_specs=pl.BlockSpec((_RB, _DP), lambda i: (i, 0)),
    out_shape=jax.ShapeDtypeStruct((_N, _DP), jnp.float32),
)


def _sage_dense(parts, xin, wl, wr, b):
    ssum = parts[0, :, :_D] + parts[1, :, :_D]
    cnt = parts[0, :, _D:_D + 1] + parts[1, :, _D:_D + 1]
    mean = ssum / jnp.maximum(cnt, 1.0)
    dn = (((1,), (1,)), ((), ()))
    acc = lax.dot_general(mean, wl, dn, preferred_element_type=jnp.float32)
    acc = acc + b
    acc = acc + lax.dot_general(xin, wr, dn, preferred_element_type=jnp.float32)
    return jnp.maximum(acc, 0.0)


def _dense_body(parts_ref, x_ref, wl_ref, wr_ref, b_ref, o_ref):
    h = _sage_dense(parts_ref[...], x_ref[...][:, :_D],
                    wl_ref[...], wr_ref[...], b_ref[...])
    o_ref[...] = jnp.concatenate([h, _ones_col(_RB)], axis=1)


_dense1 = pl.pallas_call(
    _dense_body,
    grid=(_N // _RB,),
    in_specs=[
        pl.BlockSpec((_NSC, _RB, _DP), lambda i: (0, i, 0)),
        pl.BlockSpec((_RB, _D), lambda i: (i, 0)),
        pl.BlockSpec((_D, _D), lambda i: (0, 0)),
        pl.BlockSpec((_D, _D), lambda i: (0, 0)),
        pl.BlockSpec((1, _D), lambda i: (0, 0)),
    ],
    out_specs=pl.BlockSpec((_RB, _DP), lambda i: (i, 0)),
    out_shape=jax.ShapeDtypeStruct((_N, _DP), jnp.float32),
)


def _final_body(parts_ref, h_ref, oh_ref, wl_ref, wr_ref, b_ref,
                gamma_ref, beta_ref, wout_ref, bout_ref, o_ref, pooled):
    i = pl.program_id(0)
    h2 = _sage_dense(parts_ref[...], h_ref[...][:, :_D],
                     wl_ref[...], wr_ref[...], b_ref[...])
    contrib = lax.dot_general(oh_ref[...], h2, (((0,), (0,)), ((), ())),
                              preferred_element_type=jnp.float32)

    @pl.when(i == 0)
    def _init():
        pooled[...] = jnp.zeros((_G, _D), jnp.float32)

    pooled[...] += contrib

    @pl.when(i == pl.num_programs(0) - 1)
    def _finish():
        pg = pooled[...]
        mu = jnp.mean(pg, axis=1, keepdims=True)
        var = jnp.mean((pg - mu) ** 2, axis=1, keepdims=True)
        normed = gamma_ref[...] * (pg - mu) * lax.rsqrt(var + 1e-5) \
            + beta_ref[...]
        dn = (((1,), (1,)), ((), ()))
        o_ref[...] = lax.dot_general(
            normed, wout_ref[...], dn,
            preferred_element_type=jnp.float32) + bout_ref[...]


_final = pl.pallas_call(
    _final_body,
    grid=(_N // _RB,),
    in_specs=[
        pl.BlockSpec((_NSC, _RB, _DP), lambda i: (0, i, 0)),
        pl.BlockSpec((_RB, _DP), lambda i: (i, 0)),
        pl.BlockSpec((_RB, _G), lambda i: (i, 0)),
        pl.BlockSpec((_D, _D), lambda i: (0, 0)),
        pl.BlockSpec((_D, _D), lambda i: (0, 0)),
        pl.BlockSpec((1, _D), lambda i: (0, 0)),
        pl.BlockSpec((1, _D), lambda i: (0, 0)),
        pl.BlockSpec((1, _D), lambda i: (0, 0)),
        pl.BlockSpec((_D, _D), lambda i: (0, 0)),
        pl.BlockSpec((1, _D), lambda i: (0, 0)),
    ],
    out_specs=pl.BlockSpec((_G, _D), lambda i: (0, 0)),
    out_shape=jax.ShapeDtypeStruct((_G, _D), jnp.float32),
    scratch_shapes=[pltpu.VMEM((_G, _D), jnp.float32)],
)


@jax.jit
def kernel(x, edge_index, batch, W1_l, b1_l, W1_r, W2_l, b2_l, W2_r,
           gamma, beta, W_out, b_out):
    src3 = edge_index[0].reshape(_NW, _NCH, _CH)
    dst3 = edge_index[1].reshape(_NW, _NCH, _CH)
    zeros = jnp.zeros((_RPT, _DP), jnp.float32)
    oh = (batch[:, None] == jnp.arange(_G, dtype=batch.dtype)[None, :])
    oh = oh.astype(jnp.float32)
    b1 = b1_l.reshape(1, _D)
    b2 = b2_l.reshape(1, _D)
    ga = gamma.reshape(1, _D)
    be = beta.reshape(1, _D)
    bo = b_out.reshape(1, _D)

    agg = _sc_agg()
    xpad = _pad(x)
    parts1 = agg(xpad, src3, dst3, zeros)
    h1pad = _dense1(parts1, x, W1_l, W1_r, b1)
    parts2 = agg(h1pad, src3, dst3, zeros)
    return _final(parts2, h1pad, oh, W2_l, W2_r, b2, ga, be, W_out, bo)


# speedup vs baseline: 1.6000x; 1.6000x over previous
import functools

import jax
import jax.numpy as jnp
from jax import lax
from jax.experimental import pallas as pl
from jax.experimental.pallas import tpu as pltpu
from jax.experimental.pallas import tpu_sc as plsc

_N = 10000
_E = 320000
_D = 128
_G = 64
_CW = 8
_NSC = 2
_NSUB = 16
_NW = _NSC * _NSUB
_EPW = _E // _NW
_CH = 80
_NCH = _EPW // _CH
_NPAD = 10240
_RPT = _NPAD // _NSUB
_RB = 1000


def _sc_agg1_body(table, src, dst, zeros, zerosc, ones, outp, outc,
                  srcb, dstb, rows, rows2, onesb, acc, cacc, sem, sem2):
    c = lax.axis_index("c")
    s = lax.axis_index("s")
    wid = c * _NSUB + s

    pltpu.sync_copy(src.at[wid], srcb)
    pltpu.sync_copy(dst.at[wid], dstb)
    pltpu.sync_copy(ones, onesb)
    pltpu.sync_copy(zeros, acc.at[pl.ds(s * _RPT, _RPT)])
    pltpu.sync_copy(zerosc, cacc.at[pl.ds(s * _RPT, _RPT)])
    plsc.subcore_barrier()

    pltpu.async_copy(table.at[srcb.at[0]], rows, sem)

    def chunkpair(i, carry):
        a = 2 * i
        pltpu.make_async_copy(table.at[srcb.at[a]], rows, sem).wait()
        pltpu.async_copy(table.at[srcb.at[a + 1]], rows2, sem2)
        pltpu.sync_copy(rows, acc.at[dstb.at[a]], add=True)
        pltpu.sync_copy(onesb, cacc.at[dstb.at[a]], add=True)
        pltpu.make_async_copy(table.at[srcb.at[a + 1]], rows2, sem2).wait()
        pltpu.async_copy(table.at[srcb.at[a + 2]], rows, sem)
        pltpu.sync_copy(rows2, acc.at[dstb.at[a + 1]], add=True)
        pltpu.sync_copy(onesb, cacc.at[dstb.at[a + 1]], add=True)
        return carry

    lax.fori_loop(0, (_NCH - 1) // 2, chunkpair, 0)
    a = _NCH - 1
    pltpu.make_async_copy(table.at[srcb.at[a]], rows, sem).wait()
    pltpu.sync_copy(rows, acc.at[dstb.at[a]], add=True)
    pltpu.sync_copy(onesb, cacc.at[dstb.at[a]], add=True)
    plsc.subcore_barrier()
    pltpu.sync_copy(acc.at[pl.ds(s * _RPT, _RPT)],
                    outp.at[c, pl.ds(s * _RPT, _RPT)])
    pltpu.sync_copy(cacc.at[pl.ds(s * _RPT, _RPT)],
                    outc.at[c, pl.ds(s * _RPT, _RPT)])


def _sc_agg2_body(table, src, dst, zeros, outp,
                  srcb, dstb, rows, rows2, acc, sem, sem2):
    c = lax.axis_index("c")
    s = lax.axis_index("s")
    wid = c * _NSUB + s

    pltpu.sync_copy(src.at[wid], srcb)
    pltpu.sync_copy(dst.at[wid], dstb)
    pltpu.sync_copy(zeros, acc.at[pl.ds(s * _RPT, _RPT)])
    plsc.subcore_barrier()

    pltpu.async_copy(table.at[srcb.at[0]], rows, sem)

    def chunkpair(i, carry):
        a = 2 * i
        pltpu.make_async_copy(table.at[srcb.at[a]], rows, sem).wait()
        pltpu.async_copy(table.at[srcb.at[a + 1]], rows2, sem2)
        pltpu.sync_copy(rows, acc.at[dstb.at[a]], add=True)
        pltpu.make_async_copy(table.at[srcb.at[a + 1]], rows2, sem2).wait()
        pltpu.async_copy(table.at[srcb.at[a + 2]], rows, sem)
        pltpu.sync_copy(rows2, acc.at[dstb.at[a + 1]], add=True)
        return carry

    lax.fori_loop(0, (_NCH - 1) // 2, chunkpair, 0)
    a = _NCH - 1
    pltpu.make_async_copy(table.at[srcb.at[a]], rows, sem).wait()
    pltpu.sync_copy(rows, acc.at[dstb.at[a]], add=True)
    plsc.subcore_barrier()
    pltpu.sync_copy(acc.at[pl.ds(s * _RPT, _RPT)],
                    outp.at[c, pl.ds(s * _RPT, _RPT)])


_SC_MESH = dict(core_axis_name="c", subcore_axis_name="s",
                num_cores=_NSC, num_subcores=_NSUB)


@functools.cache
def _sc_agg1():
    return pl.kernel(
        _sc_agg1_body,
        out_type=(
            jax.ShapeDtypeStruct((_NSC, _NPAD, _D), jnp.float32),
            jax.ShapeDtypeStruct((_NSC, _NPAD, _CW), jnp.float32),
        ),
        mesh=plsc.VectorSubcoreMesh(**_SC_MESH),
        scratch_types=[
            pltpu.VMEM((_NCH, _CH), jnp.int32),
            pltpu.VMEM((_NCH, _CH), jnp.int32),
            pltpu.VMEM((_CH, _D), jnp.float32),
            pltpu.VMEM((_CH, _D), jnp.float32),
            pltpu.VMEM((_CH, _CW), jnp.float32),
            pltpu.VMEM_SHARED((_NPAD, _D), jnp.float32),
            pltpu.VMEM_SHARED((_NPAD, _CW), jnp.float32),
            pltpu.SemaphoreType.DMA,
            pltpu.SemaphoreType.DMA,
        ],
        compiler_params=pltpu.CompilerParams(use_tc_tiling_on_sc=False),
    )


@functools.cache
def _sc_agg2():
    return pl.kernel(
        _sc_agg2_body,
        out_type=jax.ShapeDtypeStruct((_NSC, _NPAD, _D), jnp.float32),
        mesh=plsc.VectorSubcoreMesh(**_SC_MESH),
        scratch_types=[
            pltpu.VMEM((_NCH, _CH), jnp.int32),
            pltpu.VMEM((_NCH, _CH), jnp.int32),
            pltpu.VMEM((_CH, _D), jnp.float32),
            pltpu.VMEM((_CH, _D), jnp.float32),
            pltpu.VMEM_SHARED((_NPAD, _D), jnp.float32),
            pltpu.SemaphoreType.DMA,
            pltpu.SemaphoreType.DMA,
        ],
        compiler_params=pltpu.CompilerParams(use_tc_tiling_on_sc=False),
    )


def _sage_dense(parts, cnts, xin, wl, wr, b):
    ssum = parts[0] + parts[1]
    cnt = cnts[0, :, :1] + cnts[1, :, :1]
    mean = ssum / jnp.maximum(cnt, 1.0)
    dn = (((1,), (1,)), ((), ()))
    acc = lax.dot_general(mean, wl, dn, preferred_element_type=jnp.float32)
    acc = acc + b
    acc = acc + lax.dot_general(xin, wr, dn, preferred_element_type=jnp.float32)
    return jnp.maximum(acc, 0.0)


def _dense_body(parts_ref, cnt_ref, x_ref, wl_ref, wr_ref, b_ref, o_ref):
    o_ref[...] = _sage_dense(parts_ref[...], cnt_ref[...], x_ref[...],
                             wl_ref[...], wr_ref[...], b_ref[...])


_dense1 = pl.pallas_call(
    _dense_body,
    grid=(_N // _RB,),
    in_specs=[
        pl.BlockSpec((_NSC, _RB, _D), lambda i: (0, i, 0)),
        pl.BlockSpec((_NSC, _RB, _CW), lambda i: (0, i, 0)),
        pl.BlockSpec((_RB, _D), lambda i: (i, 0)),
        pl.BlockSpec((_D, _D), lambda i: (0, 0)),
        pl.BlockSpec((_D, _D), lambda i: (0, 0)),
        pl.BlockSpec((1, _D), lambda i: (0, 0)),
    ],
    out_specs=pl.BlockSpec((_RB, _D), lambda i: (i, 0)),
    out_shape=jax.ShapeDtypeStruct((_N, _D), jnp.float32),
)


def _final_body(parts_ref, cnt_ref, h_ref, b3_ref, wl_ref, wr_ref, b_ref,
                gamma_ref, beta_ref, wout_ref, bout_ref, o_ref, pooled):
    i = pl.program_id(0)
    h2 = _sage_dense(parts_ref[...], cnt_ref[...], h_ref[...],
                     wl_ref[...], wr_ref[...], b_ref[...])
    brow = b3_ref[...].reshape(1, _RB)
    ohT = (lax.broadcasted_iota(jnp.int32, (_G, _RB), 0)
           == jnp.broadcast_to(brow, (_G, _RB))).astype(jnp.float32)
    contrib = jnp.dot(ohT, h2, preferred_element_type=jnp.float32)

    @pl.when(i == 0)
    def _init():
        pooled[...] = jnp.zeros((_G, _D), jnp.float32)

    pooled[...] += contrib

    @pl.when(i == pl.num_programs(0) - 1)
    def _finish():
        pg = pooled[...]
        mu = jnp.mean(pg, axis=1, keepdims=True)
        var = jnp.mean((pg - mu) ** 2, axis=1, keepdims=True)
        normed = gamma_ref[...] * (pg - mu) * lax.rsqrt(var + 1e-5) \
            + beta_ref[...]
        dn = (((1,), (1,)), ((), ()))
        o_ref[...] = lax.dot_general(
            normed, wout_ref[...], dn,
            preferred_element_type=jnp.float32) + bout_ref[...]


_final = pl.pallas_call(
    _final_body,
    grid=(_N // _RB,),
    in_specs=[
        pl.BlockSpec((_NSC, _RB, _D), lambda i: (0, i, 0)),
        pl.BlockSpec((_NSC, _RB, _CW), lambda i: (0, i, 0)),
        pl.BlockSpec((_RB, _D), lambda i: (i, 0)),
        pl.BlockSpec((1, 1, _RB), lambda i: (i, 0, 0)),
        pl.BlockSpec((_D, _D), lambda i: (0, 0)),
        pl.BlockSpec((_D, _D), lambda i: (0, 0)),
        pl.BlockSpec((1, _D), lambda i: (0, 0)),
        pl.BlockSpec((1, _D), lambda i: (0, 0)),
        pl.BlockSpec((1, _D), lambda i: (0, 0)),
        pl.BlockSpec((_D, _D), lambda i: (0, 0)),
        pl.BlockSpec((1, _D), lambda i: (0, 0)),
    ],
    out_specs=pl.BlockSpec((_G, _D), lambda i: (0, 0)),
    out_shape=jax.ShapeDtypeStruct((_G, _D), jnp.float32),
    scratch_shapes=[pltpu.VMEM((_G, _D), jnp.float32)],
)


@jax.jit
def kernel(x, edge_index, batch, W1_l, b1_l, W1_r, W2_l, b2_l, W2_r,
           gamma, beta, W_out, b_out):
    src3 = edge_index[0].reshape(_NW, _NCH, _CH)
    dst3 = edge_index[1].reshape(_NW, _NCH, _CH)
    batch3 = batch.reshape(_N // _RB, 1, _RB)
    zeros = jnp.zeros((_RPT, _D), jnp.float32)
    zerosc = jnp.zeros((_RPT, _CW), jnp.float32)
    ones = jnp.ones((_CH, _CW), jnp.float32)
    b1 = b1_l.reshape(1, _D)
    b2 = b2_l.reshape(1, _D)
    ga = gamma.reshape(1, _D)
    be = beta.reshape(1, _D)
    bo = b_out.reshape(1, _D)

    parts1, cnts = _sc_agg1()(x, src3, dst3, zeros, zerosc, ones)
    h1 = _dense1(parts1, cnts, x, W1_l, W1_r, b1)
    parts2 = _sc_agg2()(h1, src3, dst3, zeros)
    return _final(parts2, cnts, h1, batch3, W2_l, W2_r, b2, ga, be,
                  W_out, bo)


# speedup vs baseline: 1.6656x; 1.0410x over previous
import functools

import jax
import jax.numpy as jnp
from jax import lax
from jax.experimental import pallas as pl
from jax.experimental.pallas import tpu as pltpu
from jax.experimental.pallas import tpu_sc as plsc

_N = 10000
_E = 320000
_D = 128
_G = 64
_CW = 8
_NSC = 2
_NSUB = 16
_NW = _NSC * _NSUB
_EPW = _E // _NW
_CH = 80
_NCH = _EPW // _CH
_NPAD = 10000
_RPT = _NPAD // _NSUB
_RB = 2000
_CB = _RB * _CW // 128


def _sc_agg1_body(table, src, dst, zeros, zerosc, ones, outp, outc,
                  srcb, dstb, rows, rows2, onesb, acc, cacc, sem, sem2):
    c = lax.axis_index("c")
    s = lax.axis_index("s")
    wid = c * _NSUB + s

    pltpu.sync_copy(src.at[wid], srcb)
    pltpu.sync_copy(dst.at[wid], dstb)
    pltpu.sync_copy(ones, onesb)
    pltpu.sync_copy(zeros, acc.at[pl.ds(s * _RPT, _RPT)])
    pltpu.sync_copy(zerosc, cacc.at[pl.ds(s * _RPT, _RPT)])
    plsc.subcore_barrier()

    pltpu.async_copy(table.at[srcb.at[0]], rows, sem)

    def chunkpair(i, carry):
        a = 2 * i
        pltpu.make_async_copy(table.at[srcb.at[a]], rows, sem).wait()
        pltpu.async_copy(table.at[srcb.at[a + 1]], rows2, sem2)
        pltpu.sync_copy(rows, acc.at[dstb.at[a]], add=True)
        pltpu.sync_copy(onesb, cacc.at[dstb.at[a]], add=True)
        pltpu.make_async_copy(table.at[srcb.at[a + 1]], rows2, sem2).wait()
        pltpu.async_copy(table.at[srcb.at[a + 2]], rows, sem)
        pltpu.sync_copy(rows2, acc.at[dstb.at[a + 1]], add=True)
        pltpu.sync_copy(onesb, cacc.at[dstb.at[a + 1]], add=True)
        return carry

    lax.fori_loop(0, (_NCH - 1) // 2, chunkpair, 0)
    a = _NCH - 1
    pltpu.make_async_copy(table.at[srcb.at[a]], rows, sem).wait()
    pltpu.sync_copy(rows, acc.at[dstb.at[a]], add=True)
    pltpu.sync_copy(onesb, cacc.at[dstb.at[a]], add=True)
    plsc.subcore_barrier()
    pltpu.sync_copy(acc.at[pl.ds(s * _RPT, _RPT)],
                    outp.at[c, pl.ds(s * _RPT, _RPT)])
    pltpu.sync_copy(cacc.at[pl.ds(s * _RPT, _RPT)],
                    outc.at[c, pl.ds(s * _RPT, _RPT)])


def _sc_agg2_body(table, src, dst, zeros, outp,
                  srcb, dstb, rows, rows2, acc, sem, sem2):
    c = lax.axis_index("c")
    s = lax.axis_index("s")
    wid = c * _NSUB + s

    pltpu.sync_copy(src.at[wid], srcb)
    pltpu.sync_copy(dst.at[wid], dstb)
    pltpu.sync_copy(zeros, acc.at[pl.ds(s * _RPT, _RPT)])
    plsc.subcore_barrier()

    pltpu.async_copy(table.at[srcb.at[0]], rows, sem)

    def chunkpair(i, carry):
        a = 2 * i
        pltpu.make_async_copy(table.at[srcb.at[a]], rows, sem).wait()
        pltpu.async_copy(table.at[srcb.at[a + 1]], rows2, sem2)
        pltpu.sync_copy(rows, acc.at[dstb.at[a]], add=True)
        pltpu.make_async_copy(table.at[srcb.at[a + 1]], rows2, sem2).wait()
        pltpu.async_copy(table.at[srcb.at[a + 2]], rows, sem)
        pltpu.sync_copy(rows2, acc.at[dstb.at[a + 1]], add=True)
        return carry

    lax.fori_loop(0, (_NCH - 1) // 2, chunkpair, 0)
    a = _NCH - 1
    pltpu.make_async_copy(table.at[srcb.at[a]], rows, sem).wait()
    pltpu.sync_copy(rows, acc.at[dstb.at[a]], add=True)
    plsc.subcore_barrier()
    pltpu.sync_copy(acc.at[pl.ds(s * _RPT, _RPT)],
                    outp.at[c, pl.ds(s * _RPT, _RPT)])


_SC_MESH = dict(core_axis_name="c", subcore_axis_name="s",
                num_cores=_NSC, num_subcores=_NSUB)


@functools.cache
def _sc_agg1():
    return pl.kernel(
        _sc_agg1_body,
        out_type=(
            jax.ShapeDtypeStruct((_NSC, _NPAD, _D), jnp.float32),
            jax.ShapeDtypeStruct((_NSC, _NPAD, _CW), jnp.float32),
        ),
        mesh=plsc.VectorSubcoreMesh(**_SC_MESH),
        scratch_types=[
            pltpu.VMEM((_NCH, _CH), jnp.int32),
            pltpu.VMEM((_NCH, _CH), jnp.int32),
            pltpu.VMEM((_CH, _D), jnp.float32),
            pltpu.VMEM((_CH, _D), jnp.float32),
            pltpu.VMEM((_CH, _CW), jnp.float32),
            pltpu.VMEM_SHARED((_NPAD, _D), jnp.float32),
            pltpu.VMEM_SHARED((_NPAD, _CW), jnp.float32),
            pltpu.SemaphoreType.DMA,
            pltpu.SemaphoreType.DMA,
        ],
        compiler_params=pltpu.CompilerParams(use_tc_tiling_on_sc=False),
    )


@functools.cache
def _sc_agg2():
    return pl.kernel(
        _sc_agg2_body,
        out_type=jax.ShapeDtypeStruct((_NSC, _NPAD, _D), jnp.float32),
        mesh=plsc.VectorSubcoreMesh(**_SC_MESH),
        scratch_types=[
            pltpu.VMEM((_NCH, _CH), jnp.int32),
            pltpu.VMEM((_NCH, _CH), jnp.int32),
            pltpu.VMEM((_CH, _D), jnp.float32),
            pltpu.VMEM((_CH, _D), jnp.float32),
            pltpu.VMEM_SHARED((_NPAD, _D), jnp.float32),
            pltpu.SemaphoreType.DMA,
            pltpu.SemaphoreType.DMA,
        ],
        compiler_params=pltpu.CompilerParams(use_tc_tiling_on_sc=False),
    )


def _sage_dense(parts, cnts, xin, wl, wr, b):
    ssum = parts[0] + parts[1]
    inv_small = 1.0 / jnp.maximum(cnts[0, 0] + cnts[1, 0], 1.0)
    pick = (lax.broadcasted_iota(jnp.int32, (_RB, _CB), 0) // 16
            == lax.broadcasted_iota(jnp.int32, (_RB, _CB), 1))
    rows = jnp.dot(pick.astype(jnp.float32), inv_small,
                   preferred_element_type=jnp.float32)
    lane = (lax.broadcasted_iota(jnp.int32, (_RB, 128), 1)
            == _CW * (lax.broadcasted_iota(jnp.int32, (_RB, 128), 0) % 16))
    inv = jnp.sum(jnp.where(lane, rows, 0.0), axis=1, keepdims=True)
    mean = ssum * inv
    dn = (((1,), (1,)), ((), ()))
    acc = lax.dot_general(mean, wl, dn, preferred_element_type=jnp.float32)
    acc = acc + b
    acc = acc + lax.dot_general(xin, wr, dn, preferred_element_type=jnp.float32)
    return jnp.maximum(acc, 0.0)


def _dense_body(parts_ref, cnt_ref, x_ref, wl_ref, wr_ref, b_ref, o_ref):
    o_ref[...] = _sage_dense(parts_ref[...], cnt_ref[...], x_ref[...],
                             wl_ref[...], wr_ref[...], b_ref[...])


_dense1 = pl.pallas_call(
    _dense_body,
    grid=(_N // _RB,),
    in_specs=[
        pl.BlockSpec((_NSC, _RB, _D), lambda i: (0, i, 0)),
        pl.BlockSpec((_NSC, 1, _CB, 128), lambda i: (0, i, 0, 0)),
        pl.BlockSpec((_RB, _D), lambda i: (i, 0)),
        pl.BlockSpec((_D, _D), lambda i: (0, 0)),
        pl.BlockSpec((_D, _D), lambda i: (0, 0)),
        pl.BlockSpec((1, _D), lambda i: (0, 0)),
    ],
    out_specs=pl.BlockSpec((_RB, _D), lambda i: (i, 0)),
    out_shape=jax.ShapeDtypeStruct((_N, _D), jnp.float32),
)


def _final_body(parts_ref, cnt_ref, h_ref, b3_ref, wl_ref, wr_ref, b_ref,
                gamma_ref, beta_ref, wout_ref, bout_ref, o_ref, pooled):
    i = pl.program_id(0)
    h2 = _sage_dense(parts_ref[...], cnt_ref[...], h_ref[...],
                     wl_ref[...], wr_ref[...], b_ref[...])
    brow = b3_ref[...].reshape(1, _RB)
    ohT = (lax.broadcasted_iota(jnp.int32, (_G, _RB), 0)
           == jnp.broadcast_to(brow, (_G, _RB))).astype(jnp.float32)
    contrib = jnp.dot(ohT, h2, preferred_element_type=jnp.float32)

    @pl.when(i == 0)
    def _init():
        pooled[...] = jnp.zeros((_G, _D), jnp.float32)

    pooled[...] += contrib

    @pl.when(i == pl.num_programs(0) - 1)
    def _finish():
        pg = pooled[...]
        mu = jnp.mean(pg, axis=1, keepdims=True)
        var = jnp.mean((pg - mu) ** 2, axis=1, keepdims=True)
        normed = gamma_ref[...] * (pg - mu) * lax.rsqrt(var + 1e-5) \
            + beta_ref[...]
        dn = (((1,), (1,)), ((), ()))
        o_ref[...] = lax.dot_general(
            normed, wout_ref[...], dn,
            preferred_element_type=jnp.float32) + bout_ref[...]


_final = pl.pallas_call(
    _final_body,
    grid=(_N // _RB,),
    in_specs=[
        pl.BlockSpec((_NSC, _RB, _D), lambda i: (0, i, 0)),
        pl.BlockSpec((_NSC, 1, _CB, 128), lambda i: (0, i, 0, 0)),
        pl.BlockSpec((_RB, _D), lambda i: (i, 0)),
        pl.BlockSpec((1, 1, _RB), lambda i: (i, 0, 0)),
        pl.BlockSpec((_D, _D), lambda i: (0, 0)),
        pl.BlockSpec((_D, _D), lambda i: (0, 0)),
        pl.BlockSpec((1, _D), lambda i: (0, 0)),
        pl.BlockSpec((1, _D), lambda i: (0, 0)),
        pl.BlockSpec((1, _D), lambda i: (0, 0)),
        pl.BlockSpec((_D, _D), lambda i: (0, 0)),
        pl.BlockSpec((1, _D), lambda i: (0, 0)),
    ],
    out_specs=pl.BlockSpec((_G, _D), lambda i: (0, 0)),
    out_shape=jax.ShapeDtypeStruct((_G, _D), jnp.float32),
    scratch_shapes=[pltpu.VMEM((_G, _D), jnp.float32)],
)


@jax.jit
def kernel(x, edge_index, batch, W1_l, b1_l, W1_r, W2_l, b2_l, W2_r,
           gamma, beta, W_out, b_out):
    src3 = edge_index[0].reshape(_NW, _NCH, _CH)
    dst3 = edge_index[1].reshape(_NW, _NCH, _CH)
    batch3 = batch.reshape(_N // _RB, 1, _RB)
    zeros = jnp.zeros((_RPT, _D), jnp.float32)
    zerosc = jnp.zeros((_RPT, _CW), jnp.float32)
    ones = jnp.ones((_CH, _CW), jnp.float32)
    b1 = b1_l.reshape(1, _D)
    b2 = b2_l.reshape(1, _D)
    ga = gamma.reshape(1, _D)
    be = beta.reshape(1, _D)
    bo = b_out.reshape(1, _D)

    parts1, cnts = _sc_agg1()(x, src3, dst3, zeros, zerosc, ones)
    cnt_view = cnts.reshape(_NSC, _N // _RB, _CB, 128)
    h1 = _dense1(parts1, cnt_view, x, W1_l, W1_r, b1)
    parts2 = _sc_agg2()(h1, src3, dst3, zeros)
    return _final(parts2, cnt_view, h1, batch3, W2_l, W2_r, b2, ga, be,
                  W_out, bo)


# speedup vs baseline: 2.0066x; 1.2047x over previous
import functools

import jax
import jax.numpy as jnp
from jax import lax
from jax.experimental import pallas as pl
from jax.experimental.pallas import tpu as pltpu
from jax.experimental.pallas import tpu_sc as plsc

_N = 10000
_E = 320000
_D = 128
_G = 64
_CW = 8
_NSC = 2
_NSUB = 16
_NW = _NSC * _NSUB
_CH = 128
_NT = _E // _CH
_TPW = _NT // _NW
_STG = 40
_NPAD = 10000
_RPT = _NPAD // _NSUB
_RB = 2000
_CB = _RB * _CW // 128


def _edge_pipeline(table, ev, rows, rows2, acc, sem, sem2, cacc, onesb,
                   first, count):
    def scat(buf, a):
        pltpu.sync_copy(buf, acc.at[ev.at[a, 1]], add=True)
        if cacc is not None:
            pltpu.sync_copy(onesb, cacc.at[ev.at[a, 1]], add=True)

    pltpu.async_copy(table.at[ev.at[first, 0]], rows, sem)

    def pairbody(i, carry):
        a = first + 2 * i
        pltpu.make_async_copy(table.at[ev.at[a, 0]], rows, sem).wait()
        pltpu.async_copy(table.at[ev.at[a + 1, 0]], rows2, sem2)
        scat(rows, a)
        pltpu.make_async_copy(table.at[ev.at[a + 1, 0]], rows2, sem2).wait()
        pltpu.async_copy(table.at[ev.at[a + 2, 0]], rows, sem)
        scat(rows2, a + 1)
        return carry

    lax.fori_loop(0, count // 2 - 1, pairbody, 0)
    a = first + count - 2
    pltpu.make_async_copy(table.at[ev.at[a, 0]], rows, sem).wait()
    pltpu.async_copy(table.at[ev.at[a + 1, 0]], rows2, sem2)
    scat(rows, a)
    pltpu.make_async_copy(table.at[ev.at[a + 1, 0]], rows2, sem2).wait()
    scat(rows2, a + 1)


def _sc_agg_common(table, et, rows, rows2, ev, acc, sem, sem2,
                   cacc=None, onesb=None):
    c = lax.axis_index("c")
    s = lax.axis_index("s")
    wid = c * _NSUB + s
    s0 = _TPW * wid

    pltpu.sync_copy(et.at[pl.ds(s0, _STG)], ev)
    _edge_pipeline(table, ev, rows, rows2, acc, sem, sem2, cacc, onesb,
                   0, _STG)
    pltpu.sync_copy(et.at[pl.ds(s0 + _TPW - _STG, _STG)], ev)
    _edge_pipeline(table, ev, rows, rows2, acc, sem, sem2, cacc, onesb,
                   2 * _STG - _TPW, _TPW - _STG)

    @pl.when(wid < _NT - _TPW * _NW)
    def _remainder():
        pltpu.sync_copy(et.at[pl.ds(_TPW * _NW + wid, 1)],
                        ev.at[pl.ds(0, 1)])
        pltpu.async_copy(table.at[ev.at[0, 0]], rows, sem).wait()
        pltpu.sync_copy(rows, acc.at[ev.at[0, 1]], add=True)
        if cacc is not None:
            pltpu.sync_copy(onesb, cacc.at[ev.at[0, 1]], add=True)


def _sc_agg1_body(table, et, zeros, zerosc, ones, outp, outc,
                  ev, rows, rows2, onesb, acc, cacc, sem, sem2):
    c = lax.axis_index("c")
    s = lax.axis_index("s")
    pltpu.sync_copy(ones, onesb)
    pltpu.sync_copy(zeros, acc.at[pl.ds(s * _RPT, _RPT)])
    pltpu.sync_copy(zerosc, cacc.at[pl.ds(s * _RPT, _RPT)])
    plsc.subcore_barrier()
    _sc_agg_common(table, et, rows, rows2, ev, acc, sem, sem2, cacc, onesb)
    plsc.subcore_barrier()
    pltpu.sync_copy(acc.at[pl.ds(s * _RPT, _RPT)],
                    outp.at[c, pl.ds(s * _RPT, _RPT)])
    pltpu.sync_copy(cacc.at[pl.ds(s * _RPT, _RPT)],
                    outc.at[c, pl.ds(s * _RPT, _RPT)])


def _sc_agg2_body(table, et, zeros, outp, ev, rows, rows2, acc, sem, sem2):
    c = lax.axis_index("c")
    s = lax.axis_index("s")
    pltpu.sync_copy(zeros, acc.at[pl.ds(s * _RPT, _RPT)])
    plsc.subcore_barrier()
    _sc_agg_common(table, et, rows, rows2, ev, acc, sem, sem2)
    plsc.subcore_barrier()
    pltpu.sync_copy(acc.at[pl.ds(s * _RPT, _RPT)],
                    outp.at[c, pl.ds(s * _RPT, _RPT)])


_SC_MESH = dict(core_axis_name="c", subcore_axis_name="s",
                num_cores=_NSC, num_subcores=_NSUB)


@functools.cache
def _sc_agg1():
    return pl.kernel(
        _sc_agg1_body,
        out_type=(
            jax.ShapeDtypeStruct((_NSC, _NPAD, _D), jnp.float32),
            jax.ShapeDtypeStruct((_NSC, _NPAD, _CW), jnp.float32),
        ),
        mesh=plsc.VectorSubcoreMesh(**_SC_MESH),
        scratch_types=[
            pltpu.VMEM((_STG, 2, _CH), jnp.int32),
            pltpu.VMEM((_CH, _D), jnp.float32),
            pltpu.VMEM((_CH, _D), jnp.float32),
            pltpu.VMEM((_CH, _CW), jnp.float32),
            pltpu.VMEM_SHARED((_NPAD, _D), jnp.float32),
            pltpu.VMEM_SHARED((_NPAD, _CW), jnp.float32),
            pltpu.SemaphoreType.DMA,
            pltpu.SemaphoreType.DMA,
        ],
        compiler_params=pltpu.CompilerParams(use_tc_tiling_on_sc=False),
    )


@functools.cache
def _sc_agg2():
    return pl.kernel(
        _sc_agg2_body,
        out_type=jax.ShapeDtypeStruct((_NSC, _NPAD, _D), jnp.float32),
        mesh=plsc.VectorSubcoreMesh(**_SC_MESH),
        scratch_types=[
            pltpu.VMEM((_STG, 2, _CH), jnp.int32),
            pltpu.VMEM((_CH, _D), jnp.float32),
            pltpu.VMEM((_CH, _D), jnp.float32),
            pltpu.VMEM_SHARED((_NPAD, _D), jnp.float32),
            pltpu.SemaphoreType.DMA,
            pltpu.SemaphoreType.DMA,
        ],
        compiler_params=pltpu.CompilerParams(use_tc_tiling_on_sc=False),
    )


def _sage_dense(parts, cnts, xin, wl, wr, b):
    ssum = parts[0] + parts[1]
    inv_small = 1.0 / jnp.maximum(cnts[0, 0] + cnts[1, 0], 1.0)
    pick = (lax.broadcasted_iota(jnp.int32, (_RB, _CB), 0) // 16
            == lax.broadcasted_iota(jnp.int32, (_RB, _CB), 1))
    rows = jnp.dot(pick.astype(jnp.float32), inv_small,
                   preferred_element_type=jnp.float32)
    lane = (lax.broadcasted_iota(jnp.int32, (_RB, 128), 1)
            == _CW * (lax.broadcasted_iota(jnp.int32, (_RB, 128), 0) % 16))
    inv = jnp.sum(jnp.where(lane, rows, 0.0), axis=1, keepdims=True)
    mean = ssum * inv
    dn = (((1,), (1,)), ((), ()))
    acc = lax.dot_general(mean, wl, dn, preferred_element_type=jnp.float32)
    acc = acc + b
    acc = acc + lax.dot_general(xin, wr, dn, preferred_element_type=jnp.float32)
    return jnp.maximum(acc, 0.0)


def _dense_body(parts_ref, cnt_ref, x_ref, wl_ref, wr_ref, b_ref, o_ref):
    o_ref[...] = _sage_dense(parts_ref[...], cnt_ref[...], x_ref[...],
                             wl_ref[...], wr_ref[...], b_ref[...])


_dense1 = pl.pallas_call(
    _dense_body,
    grid=(_N // _RB,),
    in_specs=[
        pl.BlockSpec((_NSC, _RB, _D), lambda i: (0, i, 0)),
        pl.BlockSpec((_NSC, 1, _CB, 128), lambda i: (0, i, 0, 0)),
        pl.BlockSpec((_RB, _D), lambda i: (i, 0)),
        pl.BlockSpec((_D, _D), lambda i: (0, 0)),
        pl.BlockSpec((_D, _D), lambda i: (0, 0)),
        pl.BlockSpec((1, _D), lambda i: (0, 0)),
    ],
    out_specs=pl.BlockSpec((_RB, _D), lambda i: (i, 0)),
    out_shape=jax.ShapeDtypeStruct((_N, _D), jnp.float32),
)


def _final_body(parts_ref, cnt_ref, h_ref, b3_ref, wl_ref, wr_ref, b_ref,
                gamma_ref, beta_ref, wout_ref, bout_ref, o_ref, pooled):
    i = pl.program_id(0)
    h2 = _sage_dense(parts_ref[...], cnt_ref[...], h_ref[...],
                     wl_ref[...], wr_ref[...], b_ref[...])
    brow = b3_ref[...].reshape(1, _RB)
    ohT = (lax.broadcasted_iota(jnp.int32, (_G, _RB), 0)
           == jnp.broadcast_to(brow, (_G, _RB))).astype(jnp.float32)
    contrib = jnp.dot(ohT, h2, preferred_element_type=jnp.float32)

    @pl.when(i == 0)
    def _init():
        pooled[...] = jnp.zeros((_G, _D), jnp.float32)

    pooled[...] += contrib

    @pl.when(i == pl.num_programs(0) - 1)
    def _finish():
        pg = pooled[...]
        mu = jnp.mean(pg, axis=1, keepdims=True)
        var = jnp.mean((pg - mu) ** 2, axis=1, keepdims=True)
        normed = gamma_ref[...] * (pg - mu) * lax.rsqrt(var + 1e-5) \
            + beta_ref[...]
        dn = (((1,), (1,)), ((), ()))
        o_ref[...] = lax.dot_general(
            normed, wout_ref[...], dn,
            preferred_element_type=jnp.float32) + bout_ref[...]


_final = pl.pallas_call(
    _final_body,
    grid=(_N // _RB,),
    in_specs=[
        pl.BlockSpec((_NSC, _RB, _D), lambda i: (0, i, 0)),
        pl.BlockSpec((_NSC, 1, _CB, 128), lambda i: (0, i, 0, 0)),
        pl.BlockSpec((_RB, _D), lambda i: (i, 0)),
        pl.BlockSpec((1, 1, _RB), lambda i: (i, 0, 0)),
        pl.BlockSpec((_D, _D), lambda i: (0, 0)),
        pl.BlockSpec((_D, _D), lambda i: (0, 0)),
        pl.BlockSpec((1, _D), lambda i: (0, 0)),
        pl.BlockSpec((1, _D), lambda i: (0, 0)),
        pl.BlockSpec((1, _D), lambda i: (0, 0)),
        pl.BlockSpec((_D, _D), lambda i: (0, 0)),
        pl.BlockSpec((1, _D), lambda i: (0, 0)),
    ],
    out_specs=pl.BlockSpec((_G, _D), lambda i: (0, 0)),
    out_shape=jax.ShapeDtypeStruct((_G, _D), jnp.float32),
    scratch_shapes=[pltpu.VMEM((_G, _D), jnp.float32)],
)


@jax.jit
def kernel(x, edge_index, batch, W1_l, b1_l, W1_r, W2_l, b2_l, W2_r,
           gamma, beta, W_out, b_out):
    et = edge_index.reshape(2, _NT, _CH).transpose(1, 0, 2)
    batch3 = batch.reshape(_N // _RB, 1, _RB)
    zeros = jnp.zeros((_RPT, _D), jnp.float32)
    zerosc = jnp.zeros((_RPT, _CW), jnp.float32)
    ones = jnp.ones((_CH, _CW), jnp.float32)
    b1 = b1_l.reshape(1, _D)
    b2 = b2_l.reshape(1, _D)
    ga = gamma.reshape(1, _D)
    be = beta.reshape(1, _D)
    bo = b_out.reshape(1, _D)

    parts1, cnts = _sc_agg1()(x, et, zeros, zerosc, ones)
    cnt_view = cnts.reshape(_NSC, _N // _RB, _CB, 128)
    h1 = _dense1(parts1, cnt_view, x, W1_l, W1_r, b1)
    parts2 = _sc_agg2()(h1, et, zeros)
    return _final(parts2, cnt_view, h1, batch3, W2_l, W2_r, b2, ga, be,
                  W_out, bo)
